# Initial kernel scaffold; baseline (speedup 1.0000x reference)
import jax
import jax.numpy as jnp
from jax.experimental import pallas as pl


def kernel(src_escrito_por, dst_escrito_por, src_escreveu, dst_escreveu, src_tem_genero, dst_tem_genero, src_pertence_a, dst_pertence_a, Wl1_e1, b1_e1, Wr1_e1, Wl1_e2, b1_e2, Wr1_e2, Wl1_e3, b1_e3, Wr1_e3, Wl1_e4, b1_e4, Wr1_e4, Wl2_e1, b2_e1, Wr2_e1, Wl2_e2, b2_e2, Wr2_e2, Wl2_e3, b2_e3, Wr2_e3, Wl2_e4, b2_e4, Wr2_e4):
    raise NotImplementedError("write your pallas kernel here")


# speedup vs baseline: 5.4843x; 5.4843x over previous
import jax
import jax.numpy as jnp
from jax import lax
from jax.experimental import pallas as pl
from jax.experimental.pallas import tpu as pltpu
from jax.experimental.pallas import tpu_sc as plsc

NL, NA, NG = 5000, 2500, 100
NE = 10000
HID, OUTD = 256, 128
HH = 128
LP, AP, GP = 5120, 2560, 128
EP = 10240
NT = 16
EPT = EP // NT
CH = 64
NCH = EPT // CH
ZR = 80
CW = 16
F32 = jnp.float32

_MESH = plsc.VectorSubcoreMesh(core_axis_name="c", subcore_axis_name="s")

_SEG_SCRATCH = [
    pltpu.VMEM((CH,), jnp.int32),
    pltpu.VMEM((CH,), jnp.int32),
    pltpu.VMEM((CH, HH), F32),
    pltpu.VMEM((ZR, HH), F32),
    pltpu.VMEM((CH, HH), F32),
    pltpu.VMEM_SHARED((LP, HH), F32),
    pltpu.SemaphoreType.DMA,
]
_SEG1_SCRATCH = _SEG_SCRATCH[:-1] + [
    pltpu.VMEM_SHARED((LP, HH), F32),
    pltpu.SemaphoreType.DMA,
]


def _phase_maker(idx_v, dst_v, rows_v, zrow_v, ones_v, acc, cnt, sem,
                 cid, tid):
    def phase(src_h, dst_h, tab_h, out_h, cnt_h, rpt, counts):
        rbase = tid * rpt
        for z in range(rpt // ZR):
            pltpu.sync_copy(zrow_v, acc.at[pl.ds(rbase + z * ZR, ZR)])
            if counts:
                pltpu.sync_copy(zrow_v, cnt.at[pl.ds(rbase + z * ZR, ZR)])
        plsc.subcore_barrier()
        ebase = cid * EP + tid * EPT
        for k in range(NCH):
            off = ebase + k * CH
            pltpu.sync_copy(src_h.at[pl.ds(off, CH)], idx_v)
            pltpu.sync_copy(dst_h.at[pl.ds(off, CH)], dst_v)
            pltpu.async_copy(tab_h.at[idx_v], rows_v, sem).wait()
            pltpu.sync_copy(rows_v, acc.at[dst_v], add=True)
            if counts:
                pltpu.sync_copy(ones_v, cnt.at[dst_v], add=True)
        plsc.subcore_barrier()
        obase = cid * rpt * NT + rbase
        pltpu.sync_copy(acc.at[pl.ds(rbase, rpt)],
                        out_h.at[pl.ds(obase, rpt)])
        if counts:
            pltpu.sync_copy(cnt.at[pl.ds(rbase, rpt)],
                            cnt_h.at[pl.ds(obase, rpt)])
        plsc.subcore_barrier()

    return phase


def _seg1_kernel():
    out_type = (
        jax.ShapeDtypeStruct((2 * LP, HH), F32),
        jax.ShapeDtypeStruct((2 * LP, HH), F32),
        jax.ShapeDtypeStruct((2 * AP, HH), F32),
        jax.ShapeDtypeStruct((2 * AP, HH), F32),
        jax.ShapeDtypeStruct((2 * LP, HH), F32),
        jax.ShapeDtypeStruct((2 * AP, HH), F32),
    )

    def body(src_b, dst_b, src_s, dst_s, tab_bl, tab_bh, tab_sl, tab_sh,
             zrow_h, ones_h,
             out_bl, out_bh, out_sl, out_sh, cnt_b, cnt_s,
             idx_v, dst_v, rows_v, zrow_v, ones_v, acc, cnt, sem):
        cid = lax.axis_index("c")
        tid = lax.axis_index("s")
        pltpu.sync_copy(zrow_h, zrow_v)
        pltpu.sync_copy(ones_h, ones_v)
        phase = _phase_maker(idx_v, dst_v, rows_v, zrow_v, ones_v,
                             acc, cnt, sem, cid, tid)
        phase(src_b, dst_b, tab_bl, out_bl, cnt_b, LP // NT, True)
        phase(src_b, dst_b, tab_bh, out_bh, None, LP // NT, False)
        phase(src_s, dst_s, tab_sl, out_sl, cnt_s, AP // NT, True)
        phase(src_s, dst_s, tab_sh, out_sh, None, AP // NT, False)

    return pl.kernel(body, out_type=out_type, mesh=_MESH,
                     scratch_types=_SEG1_SCRATCH)


def _seg2_kernel():
    out_type = (
        jax.ShapeDtypeStruct((2 * LP, HH), F32),
        jax.ShapeDtypeStruct((2 * AP, HH), F32),
    )

    def body(src_b, dst_b, src_s, dst_s, tab_b, tab_s, zrow_h,
             out_b, out_s,
             idx_v, dst_v, rows_v, zrow_v, ones_v, acc, sem):
        cid = lax.axis_index("c")
        tid = lax.axis_index("s")
        pltpu.sync_copy(zrow_h, zrow_v)
        phase = _phase_maker(idx_v, dst_v, rows_v, zrow_v, ones_v,
                             acc, None, sem, cid, tid)
        phase(src_b, dst_b, tab_b, out_b, None, LP // NT, False)
        phase(src_s, dst_s, tab_s, out_s, None, AP // NT, False)

    return pl.kernel(body, out_type=out_type, mesh=_MESH,
                     scratch_types=_SEG_SCRATCH)


def _tc1_livro(s2l, s2h, c2, s4l, s4h, c4, w2, w4, b2, b4, wl1, wl3, wr2, wr4):
    BLK = 256

    def body(s2lr, s2hr, c2r, s4lr, s4hr, c4r, w2r, w4r, b2r, b4r,
             a1r, a3r, r2r, r4r, p1o, p3o, rlo):
        cc2 = jnp.maximum(c2r[:, 0:1], 1.0)
        cc4 = jnp.maximum(c4r[:, 0:1], 1.0)
        s2 = jnp.concatenate([s2lr[...], s2hr[...]], axis=1)
        s4 = jnp.concatenate([s4lr[...], s4hr[...]], axis=1)
        h = (s2 / cc2 + b2r[...] + w2r[...] + s4 / cc4 + b4r[...] + w4r[...]) * 0.5
        h = jnp.maximum(h, 0.0)
        p1o[...] = jnp.dot(h, a1r[...], preferred_element_type=F32)
        p3o[...] = jnp.dot(h, a3r[...], preferred_element_type=F32)
        rlo[...] = jnp.dot(h, r2r[...] + r4r[...], preferred_element_type=F32)

    row = lambda i: (i, 0)
    fix = lambda i: (0, 0)
    return pl.pallas_call(
        body,
        grid=(LP // BLK,),
        in_specs=[
            pl.BlockSpec((BLK, HH), row), pl.BlockSpec((BLK, HH), row),
            pl.BlockSpec((BLK, HH), row),
            pl.BlockSpec((BLK, HH), row), pl.BlockSpec((BLK, HH), row),
            pl.BlockSpec((BLK, HH), row),
            pl.BlockSpec((BLK, HID), row), pl.BlockSpec((BLK, HID), row),
            pl.BlockSpec((1, HID), fix), pl.BlockSpec((1, HID), fix),
            pl.BlockSpec((HID, OUTD), fix), pl.BlockSpec((HID, OUTD), fix),
            pl.BlockSpec((HID, OUTD), fix), pl.BlockSpec((HID, OUTD), fix),
        ],
        out_specs=[pl.BlockSpec((BLK, OUTD), row)] * 3,
        out_shape=[jax.ShapeDtypeStruct((LP, OUTD), F32)] * 3,
    )(s2l, s2h, c2, s4l, s4h, c4, w2, w4, b2, b4, wl1, wl3, wr2, wr4)


def _tc1_single(s1l, s1h, c1, w1, b1, wl, wr, rows, blk):
    def body(s1lr, s1hr, c1r, w1r, b1r, wlr, wrr, po, ro):
        cc = jnp.maximum(c1r[:, 0:1], 1.0)
        s1 = jnp.concatenate([s1lr[...], s1hr[...]], axis=1)
        h = jnp.maximum(s1 / cc + b1r[...] + w1r[...], 0.0)
        po[...] = jnp.dot(h, wlr[...], preferred_element_type=F32)
        ro[...] = jnp.dot(h, wrr[...], preferred_element_type=F32)

    row = lambda i: (i, 0)
    fix = lambda i: (0, 0)
    return pl.pallas_call(
        body,
        grid=(rows // blk,),
        in_specs=[
            pl.BlockSpec((blk, HH), row), pl.BlockSpec((blk, HH), row),
            pl.BlockSpec((blk, HH), row),
            pl.BlockSpec((blk, HID), row), pl.BlockSpec((1, HID), fix),
            pl.BlockSpec((HID, OUTD), fix), pl.BlockSpec((HID, OUTD), fix),
        ],
        out_specs=[pl.BlockSpec((blk, OUTD), row)] * 2,
        out_shape=[jax.ShapeDtypeStruct((rows, OUTD), F32)] * 2,
    )(s1l, s1h, c1, w1, b1, wl, wr)


def _l2n(v):
    n = jnp.sqrt(jnp.sum(v * v, axis=1, keepdims=True))
    return v / jnp.maximum(n, 1e-12)


def _tc2_livro(s2, c2, s4, c4, rl, b2, b4):
    BLK = 256

    def body(s2r, c2r, s4r, c4r, rlr, b2r, b4r, o):
        cc2 = jnp.maximum(c2r[:, 0:1], 1.0)
        cc4 = jnp.maximum(c4r[:, 0:1], 1.0)
        v = (s2r[...] / cc2 + s4r[...] / cc4 + b2r[...] + b4r[...]
             + rlr[...]) * 0.5
        o[...] = _l2n(v)

    row = lambda i: (i, 0)
    fix = lambda i: (0, 0)
    return pl.pallas_call(
        body,
        grid=(LP // BLK,),
        in_specs=[
            pl.BlockSpec((BLK, OUTD), row), pl.BlockSpec((BLK, HH), row),
            pl.BlockSpec((BLK, OUTD), row), pl.BlockSpec((BLK, HH), row),
            pl.BlockSpec((BLK, OUTD), row),
            pl.BlockSpec((1, OUTD), fix), pl.BlockSpec((1, OUTD), fix),
        ],
        out_specs=pl.BlockSpec((BLK, OUTD), row),
        out_shape=jax.ShapeDtypeStruct((LP, OUTD), F32),
    )(s2, c2, s4, c4, rl, b2, b4)


def _tc2_single(s1, c1, r1, b1, rows, blk):
    def body(s1r, c1r, r1r, b1r, o):
        cc = jnp.maximum(c1r[:, 0:1], 1.0)
        o[...] = _l2n(s1r[...] / cc + b1r[...] + r1r[...])

    row = lambda i: (i, 0)
    fix = lambda i: (0, 0)
    return pl.pallas_call(
        body,
        grid=(rows // blk,),
        in_specs=[
            pl.BlockSpec((blk, OUTD), row), pl.BlockSpec((blk, HH), row),
            pl.BlockSpec((blk, OUTD), row), pl.BlockSpec((1, OUTD), fix),
        ],
        out_specs=pl.BlockSpec((blk, OUTD), row),
        out_shape=jax.ShapeDtypeStruct((rows, OUTD), F32),
    )(s1, c1, r1, b1)


def kernel(src_escrito_por, dst_escrito_por, src_escreveu, dst_escreveu,
           src_tem_genero, dst_tem_genero, src_pertence_a, dst_pertence_a,
           Wl1_e1, b1_e1, Wr1_e1, Wl1_e2, b1_e2, Wr1_e2,
           Wl1_e3, b1_e3, Wr1_e3, Wl1_e4, b1_e4, Wr1_e4,
           Wl2_e1, b2_e1, Wr2_e1, Wl2_e2, b2_e2, Wr2_e2,
           Wl2_e3, b2_e3, Wr2_e3, Wl2_e4, b2_e4, Wr2_e4):
    i32 = jnp.int32

    def pad_e(a, fill):
        return jnp.concatenate([a, jnp.full((EP - NE,), fill, i32)])

    s1p, d1p = pad_e(src_escrito_por, 0), pad_e(dst_escrito_por, NA)
    s2p, d2p = pad_e(src_escreveu, 0), pad_e(dst_escreveu, NL)
    s3p, d3p = pad_e(src_tem_genero, 0), pad_e(dst_tem_genero, NG)
    s4p, d4p = pad_e(src_pertence_a, 0), pad_e(dst_pertence_a, NL)

    src_b1 = jnp.concatenate([s2p, s4p + NA])
    dst_b = jnp.concatenate([d2p, d4p])
    src_s1 = jnp.concatenate([s1p, s3p + NL])
    dst_s = jnp.concatenate([d1p, d3p])
    src_b2 = jnp.concatenate([s2p, s4p + AP])
    src_s2 = jnp.concatenate([s1p, s3p + LP])

    zrow = jnp.zeros((ZR, HH), F32)
    ones = jnp.ones((CH, HH), F32)

    tab_bl = jnp.concatenate([Wl1_e2[:, :HH], Wl1_e4[:, :HH]])
    tab_bh = jnp.concatenate([Wl1_e2[:, HH:], Wl1_e4[:, HH:]])
    tab_sl = jnp.concatenate([Wl1_e1[:, :HH], Wl1_e3[:, :HH]])
    tab_sh = jnp.concatenate([Wl1_e1[:, HH:], Wl1_e3[:, HH:]])
    out_bl, out_bh, out_sl, out_sh, cnt_b, cnt_s = _seg1_kernel()(
        src_b1, dst_b, src_s1, dst_s, tab_bl, tab_bh, tab_sl, tab_sh,
        zrow, ones)
    cnt_e2, cnt_e4 = cnt_b[:LP], cnt_b[LP:]
    cnt_e1, cnt_e3 = cnt_s[:AP], cnt_s[AP:]

    wr1_e2p = jnp.pad(Wr1_e2, ((0, LP - NL), (0, 0)))
    wr1_e4p = jnp.pad(Wr1_e4, ((0, LP - NL), (0, 0)))
    wr1_e1p = jnp.pad(Wr1_e1, ((0, AP - NA), (0, 0)))
    wr1_e3p = jnp.pad(Wr1_e3, ((0, GP - NG), (0, 0)))
    b = lambda x: x.reshape(1, -1)

    P1, P3, RL = _tc1_livro(out_bl[:LP], out_bh[:LP], cnt_e2,
                            out_bl[LP:], out_bh[LP:], cnt_e4,
                            wr1_e2p, wr1_e4p, b(b1_e2), b(b1_e4),
                            Wl2_e1, Wl2_e3, Wr2_e2, Wr2_e4)
    P2, R1 = _tc1_single(out_sl[:AP], out_sh[:AP], cnt_e1, wr1_e1p, b(b1_e1),
                         Wl2_e2, Wr2_e1, AP, 256)
    P4, R3 = _tc1_single(out_sl[AP:AP + GP], out_sh[AP:AP + GP], cnt_e3[:GP],
                         wr1_e3p, b(b1_e3), Wl2_e4, Wr2_e3, GP, GP)

    tab_b2 = jnp.concatenate([P2, P4])
    tab_s2 = jnp.concatenate([P1, P3])
    out2_b, out2_s = _seg2_kernel()(
        src_b2, dst_b, src_s2, dst_s, tab_b2, tab_s2, zrow)

    out_l = _tc2_livro(out2_b[:LP], cnt_e2, out2_b[LP:], cnt_e4, RL,
                       b(b2_e2), b(b2_e4))
    out_a = _tc2_single(out2_s[:AP], cnt_e1, R1, b(b2_e1), AP, 256)
    out_g = _tc2_single(out2_s[AP:AP + GP], cnt_e3[:GP], R3, b(b2_e3), GP, GP)

    return (out_l[:NL], out_a[:NA], out_g[:NG])


# speedup vs baseline: 6.5534x; 1.1949x over previous
import jax
import jax.numpy as jnp
from jax import lax
from jax.experimental import pallas as pl
from jax.experimental.pallas import tpu as pltpu
from jax.experimental.pallas import tpu_sc as plsc

NL, NA, NG = 5000, 2500, 100
NE = 10000
HID, OUTD = 256, 128
HH = 128
LP, AP, GP = 5120, 2560, 128
EP = 10240
NT = 16
EPT = EP // NT
CH = 64
NCH = EPT // CH
ZR = 80
CW = 16
F32 = jnp.float32

_MESH = plsc.VectorSubcoreMesh(core_axis_name="c", subcore_axis_name="s")

_SEG_SCRATCH = [
    pltpu.VMEM((CH,), jnp.int32),
    pltpu.VMEM((CH,), jnp.int32),
    pltpu.VMEM((CH,), jnp.int32),
    pltpu.VMEM((CH,), jnp.int32),
    pltpu.VMEM((CH, HH), F32),
    pltpu.VMEM((CH, HH), F32),
    pltpu.VMEM((ZR, HH), F32),
    pltpu.VMEM((CH, HH), F32),
    pltpu.VMEM_SHARED((LP, HH), F32),
    pltpu.SemaphoreType.DMA,
    pltpu.SemaphoreType.DMA,
    pltpu.SemaphoreType.DMA,
    pltpu.SemaphoreType.DMA,
]
_SEG1_SCRATCH = _SEG_SCRATCH + [
    pltpu.VMEM_SHARED((LP, HH), F32),
]


def _phase_maker(idx, dst, rows, zrow_v, ones_v, acc, cnt, gsem, ssem,
                 cid, tid):
    def phase(src_h, dst_h, tab_h, out_h, cnt_h, rpt, counts):
        rbase = tid * rpt
        for z in range(rpt // ZR):
            pltpu.sync_copy(zrow_v, acc.at[pl.ds(rbase + z * ZR, ZR)])
            if counts:
                pltpu.sync_copy(zrow_v, cnt.at[pl.ds(rbase + z * ZR, ZR)])
        plsc.subcore_barrier()
        ebase = cid * EP + tid * EPT
        gd = [None] * NCH
        sd = [None] * NCH
        pltpu.sync_copy(src_h.at[pl.ds(ebase, CH)], idx[0])
        pltpu.sync_copy(dst_h.at[pl.ds(ebase, CH)], dst[0])
        gd[0] = pltpu.async_copy(tab_h.at[idx[0]], rows[0], gsem[0])
        for k in range(NCH):
            b = k & 1
            nb = 1 - b
            if k + 1 < NCH:
                if k >= 1:
                    sd[k - 1].wait()
                off = ebase + (k + 1) * CH
                pltpu.sync_copy(src_h.at[pl.ds(off, CH)], idx[nb])
                pltpu.sync_copy(dst_h.at[pl.ds(off, CH)], dst[nb])
                gd[k + 1] = pltpu.async_copy(tab_h.at[idx[nb]], rows[nb],
                                             gsem[nb])
            gd[k].wait()
            sd[k] = pltpu.async_copy(rows[b], acc.at[dst[b]], ssem[b],
                                     add=True)
            if counts:
                pltpu.sync_copy(ones_v, cnt.at[dst[b]], add=True)
        sd[NCH - 1].wait()
        if NCH >= 2:
            sd[NCH - 2].wait()
        plsc.subcore_barrier()
        obase = cid * rpt * NT + rbase
        pltpu.sync_copy(acc.at[pl.ds(rbase, rpt)],
                        out_h.at[pl.ds(obase, rpt)])
        if counts:
            pltpu.sync_copy(cnt.at[pl.ds(rbase, rpt)],
                            cnt_h.at[pl.ds(obase, rpt)])
        plsc.subcore_barrier()

    return phase


def _seg1_kernel():
    out_type = (
        jax.ShapeDtypeStruct((2 * LP, HH), F32),
        jax.ShapeDtypeStruct((2 * LP, HH), F32),
        jax.ShapeDtypeStruct((2 * AP, HH), F32),
        jax.ShapeDtypeStruct((2 * AP, HH), F32),
        jax.ShapeDtypeStruct((2 * LP, HH), F32),
        jax.ShapeDtypeStruct((2 * AP, HH), F32),
    )

    def body(src_b, dst_b, src_s, dst_s, tab_bl, tab_bh, tab_sl, tab_sh,
             zrow_h, ones_h,
             out_bl, out_bh, out_sl, out_sh, cnt_b, cnt_s,
             idx0, idx1, dst0, dst1, rows0, rows1, zrow_v, ones_v, acc,
             gsem0, gsem1, ssem0, ssem1, cnt):
        cid = lax.axis_index("c")
        tid = lax.axis_index("s")
        pltpu.sync_copy(zrow_h, zrow_v)
        pltpu.sync_copy(ones_h, ones_v)
        phase = _phase_maker((idx0, idx1), (dst0, dst1), (rows0, rows1),
                             zrow_v, ones_v, acc, cnt,
                             (gsem0, gsem1), (ssem0, ssem1), cid, tid)
        phase(src_b, dst_b, tab_bl, out_bl, cnt_b, LP // NT, True)
        phase(src_b, dst_b, tab_bh, out_bh, None, LP // NT, False)
        phase(src_s, dst_s, tab_sl, out_sl, cnt_s, AP // NT, True)
        phase(src_s, dst_s, tab_sh, out_sh, None, AP // NT, False)

    return pl.kernel(body, out_type=out_type, mesh=_MESH,
                     scratch_types=_SEG1_SCRATCH)


def _seg2_kernel():
    out_type = (
        jax.ShapeDtypeStruct((2 * LP, HH), F32),
        jax.ShapeDtypeStruct((2 * AP, HH), F32),
    )

    def body(src_b, dst_b, src_s, dst_s, tab_b, tab_s, zrow_h,
             out_b, out_s,
             idx0, idx1, dst0, dst1, rows0, rows1, zrow_v, ones_v, acc,
             gsem0, gsem1, ssem0, ssem1):
        cid = lax.axis_index("c")
        tid = lax.axis_index("s")
        pltpu.sync_copy(zrow_h, zrow_v)
        phase = _phase_maker((idx0, idx1), (dst0, dst1), (rows0, rows1),
                             zrow_v, ones_v, acc, None,
                             (gsem0, gsem1), (ssem0, ssem1), cid, tid)
        phase(src_b, dst_b, tab_b, out_b, None, LP // NT, False)
        phase(src_s, dst_s, tab_s, out_s, None, AP // NT, False)

    return pl.kernel(body, out_type=out_type, mesh=_MESH,
                     scratch_types=_SEG_SCRATCH)


def _tc1_livro(s2l, s2h, c2, s4l, s4h, c4, w2, w4, b2, b4, wl1, wl3, wr2, wr4):
    BLK = 256

    def body(s2lr, s2hr, c2r, s4lr, s4hr, c4r, w2r, w4r, b2r, b4r,
             a1r, a3r, r2r, r4r, p1o, p3o, rlo):
        cc2 = jnp.maximum(c2r[:, 0:1], 1.0)
        cc4 = jnp.maximum(c4r[:, 0:1], 1.0)
        s2 = jnp.concatenate([s2lr[...], s2hr[...]], axis=1)
        s4 = jnp.concatenate([s4lr[...], s4hr[...]], axis=1)
        h = (s2 / cc2 + b2r[...] + w2r[...] + s4 / cc4 + b4r[...] + w4r[...]) * 0.5
        h = jnp.maximum(h, 0.0)
        p1o[...] = jnp.dot(h, a1r[...], preferred_element_type=F32)
        p3o[...] = jnp.dot(h, a3r[...], preferred_element_type=F32)
        rlo[...] = jnp.dot(h, r2r[...] + r4r[...], preferred_element_type=F32)

    row = lambda i: (i, 0)
    fix = lambda i: (0, 0)
    return pl.pallas_call(
        body,
        grid=(LP // BLK,),
        in_specs=[
            pl.BlockSpec((BLK, HH), row), pl.BlockSpec((BLK, HH), row),
            pl.BlockSpec((BLK, HH), row),
            pl.BlockSpec((BLK, HH), row), pl.BlockSpec((BLK, HH), row),
            pl.BlockSpec((BLK, HH), row),
            pl.BlockSpec((BLK, HID), row), pl.BlockSpec((BLK, HID), row),
            pl.BlockSpec((1, HID), fix), pl.BlockSpec((1, HID), fix),
            pl.BlockSpec((HID, OUTD), fix), pl.BlockSpec((HID, OUTD), fix),
            pl.BlockSpec((HID, OUTD), fix), pl.BlockSpec((HID, OUTD), fix),
        ],
        out_specs=[pl.BlockSpec((BLK, OUTD), row)] * 3,
        out_shape=[jax.ShapeDtypeStruct((LP, OUTD), F32)] * 3,
    )(s2l, s2h, c2, s4l, s4h, c4, w2, w4, b2, b4, wl1, wl3, wr2, wr4)


def _tc1_single(s1l, s1h, c1, w1, b1, wl, wr, rows, blk):
    def body(s1lr, s1hr, c1r, w1r, b1r, wlr, wrr, po, ro):
        cc = jnp.maximum(c1r[:, 0:1], 1.0)
        s1 = jnp.concatenate([s1lr[...], s1hr[...]], axis=1)
        h = jnp.maximum(s1 / cc + b1r[...] + w1r[...], 0.0)
        po[...] = jnp.dot(h, wlr[...], preferred_element_type=F32)
        ro[...] = jnp.dot(h, wrr[...], preferred_element_type=F32)

    row = lambda i: (i, 0)
    fix = lambda i: (0, 0)
    return pl.pallas_call(
        body,
        grid=(rows // blk,),
        in_specs=[
            pl.BlockSpec((blk, HH), row), pl.BlockSpec((blk, HH), row),
            pl.BlockSpec((blk, HH), row),
            pl.BlockSpec((blk, HID), row), pl.BlockSpec((1, HID), fix),
            pl.BlockSpec((HID, OUTD), fix), pl.BlockSpec((HID, OUTD), fix),
        ],
        out_specs=[pl.BlockSpec((blk, OUTD), row)] * 2,
        out_shape=[jax.ShapeDtypeStruct((rows, OUTD), F32)] * 2,
    )(s1l, s1h, c1, w1, b1, wl, wr)


def _l2n(v):
    n = jnp.sqrt(jnp.sum(v * v, axis=1, keepdims=True))
    return v / jnp.maximum(n, 1e-12)


def _tc2_livro(s2, c2, s4, c4, rl, b2, b4):
    BLK = 256

    def body(s2r, c2r, s4r, c4r, rlr, b2r, b4r, o):
        cc2 = jnp.maximum(c2r[:, 0:1], 1.0)
        cc4 = jnp.maximum(c4r[:, 0:1], 1.0)
        v = (s2r[...] / cc2 + s4r[...] / cc4 + b2r[...] + b4r[...]
             + rlr[...]) * 0.5
        o[...] = _l2n(v)

    row = lambda i: (i, 0)
    fix = lambda i: (0, 0)
    return pl.pallas_call(
        body,
        grid=(LP // BLK,),
        in_specs=[
            pl.BlockSpec((BLK, OUTD), row), pl.BlockSpec((BLK, HH), row),
            pl.BlockSpec((BLK, OUTD), row), pl.BlockSpec((BLK, HH), row),
            pl.BlockSpec((BLK, OUTD), row),
            pl.BlockSpec((1, OUTD), fix), pl.BlockSpec((1, OUTD), fix),
        ],
        out_specs=pl.BlockSpec((BLK, OUTD), row),
        out_shape=jax.ShapeDtypeStruct((LP, OUTD), F32),
    )(s2, c2, s4, c4, rl, b2, b4)


def _tc2_single(s1, c1, r1, b1, rows, blk):
    def body(s1r, c1r, r1r, b1r, o):
        cc = jnp.maximum(c1r[:, 0:1], 1.0)
        o[...] = _l2n(s1r[...] / cc + b1r[...] + r1r[...])

    row = lambda i: (i, 0)
    fix = lambda i: (0, 0)
    return pl.pallas_call(
        body,
        grid=(rows // blk,),
        in_specs=[
            pl.BlockSpec((blk, OUTD), row), pl.BlockSpec((blk, HH), row),
            pl.BlockSpec((blk, OUTD), row), pl.BlockSpec((1, OUTD), fix),
        ],
        out_specs=pl.BlockSpec((blk, OUTD), row),
        out_shape=jax.ShapeDtypeStruct((rows, OUTD), F32),
    )(s1, c1, r1, b1)


def kernel(src_escrito_por, dst_escrito_por, src_escreveu, dst_escreveu,
           src_tem_genero, dst_tem_genero, src_pertence_a, dst_pertence_a,
           Wl1_e1, b1_e1, Wr1_e1, Wl1_e2, b1_e2, Wr1_e2,
           Wl1_e3, b1_e3, Wr1_e3, Wl1_e4, b1_e4, Wr1_e4,
           Wl2_e1, b2_e1, Wr2_e1, Wl2_e2, b2_e2, Wr2_e2,
           Wl2_e3, b2_e3, Wr2_e3, Wl2_e4, b2_e4, Wr2_e4):
    i32 = jnp.int32

    def pad_e(a, fill):
        return jnp.concatenate([a, jnp.full((EP - NE,), fill, i32)])

    s1p, d1p = pad_e(src_escrito_por, 0), pad_e(dst_escrito_por, NA)
    s2p, d2p = pad_e(src_escreveu, 0), pad_e(dst_escreveu, NL)
    s3p, d3p = pad_e(src_tem_genero, 0), pad_e(dst_tem_genero, NG)
    s4p, d4p = pad_e(src_pertence_a, 0), pad_e(dst_pertence_a, NL)

    src_b1 = jnp.concatenate([s2p, s4p + NA])
    dst_b = jnp.concatenate([d2p, d4p])
    src_s1 = jnp.concatenate([s1p, s3p + NL])
    dst_s = jnp.concatenate([d1p, d3p])
    src_b2 = jnp.concatenate([s2p, s4p + AP])
    src_s2 = jnp.concatenate([s1p, s3p + LP])

    zrow = jnp.zeros((ZR, HH), F32)
    ones = jnp.ones((CH, HH), F32)

    tab_bl = jnp.concatenate([Wl1_e2[:, :HH], Wl1_e4[:, :HH]])
    tab_bh = jnp.concatenate([Wl1_e2[:, HH:], Wl1_e4[:, HH:]])
    tab_sl = jnp.concatenate([Wl1_e1[:, :HH], Wl1_e3[:, :HH]])
    tab_sh = jnp.concatenate([Wl1_e1[:, HH:], Wl1_e3[:, HH:]])
    out_bl, out_bh, out_sl, out_sh, cnt_b, cnt_s = _seg1_kernel()(
        src_b1, dst_b, src_s1, dst_s, tab_bl, tab_bh, tab_sl, tab_sh,
        zrow, ones)
    cnt_e2, cnt_e4 = cnt_b[:LP], cnt_b[LP:]
    cnt_e1, cnt_e3 = cnt_s[:AP], cnt_s[AP:]

    wr1_e2p = jnp.pad(Wr1_e2, ((0, LP - NL), (0, 0)))
    wr1_e4p = jnp.pad(Wr1_e4, ((0, LP - NL), (0, 0)))
    wr1_e1p = jnp.pad(Wr1_e1, ((0, AP - NA), (0, 0)))
    wr1_e3p = jnp.pad(Wr1_e3, ((0, GP - NG), (0, 0)))
    b = lambda x: x.reshape(1, -1)

    P1, P3, RL = _tc1_livro(out_bl[:LP], out_bh[:LP], cnt_e2,
                            out_bl[LP:], out_bh[LP:], cnt_e4,
                            wr1_e2p, wr1_e4p, b(b1_e2), b(b1_e4),
                            Wl2_e1, Wl2_e3, Wr2_e2, Wr2_e4)
    P2, R1 = _tc1_single(out_sl[:AP], out_sh[:AP], cnt_e1, wr1_e1p, b(b1_e1),
                         Wl2_e2, Wr2_e1, AP, 256)
    P4, R3 = _tc1_single(out_sl[AP:AP + GP], out_sh[AP:AP + GP], cnt_e3[:GP],
                         wr1_e3p, b(b1_e3), Wl2_e4, Wr2_e3, GP, GP)

    tab_b2 = jnp.concatenate([P2, P4])
    tab_s2 = jnp.concatenate([P1, P3])
    out2_b, out2_s = _seg2_kernel()(
        src_b2, dst_b, src_s2, dst_s, tab_b2, tab_s2, zrow)

    out_l = _tc2_livro(out2_b[:LP], cnt_e2, out2_b[LP:], cnt_e4, RL,
                       b(b2_e2), b(b2_e4))
    out_a = _tc2_single(out2_s[:AP], cnt_e1, R1, b(b2_e1), AP, 256)
    out_g = _tc2_single(out2_s[AP:AP + GP], cnt_e3[:GP], R3, b(b2_e3), GP, GP)

    return (out_l[:NL], out_a[:NA], out_g[:NG])


# speedup vs baseline: 6.8151x; 1.0399x over previous
import jax
import jax.numpy as jnp
from jax import lax
from jax.experimental import pallas as pl
from jax.experimental.pallas import tpu as pltpu
from jax.experimental.pallas import tpu_sc as plsc

NL, NA, NG = 5000, 2500, 100
NE = 10000
HID, OUTD = 256, 128
HH = 128
LP, AP, GP = 5120, 2560, 128
EP = 10240
NT = 16
EPT = EP // NT
CH = 64
NCH = EPT // CH
ZR = 80
CW = 16
F32 = jnp.float32

_MESH = plsc.VectorSubcoreMesh(core_axis_name="c", subcore_axis_name="s")

_SEG_SCRATCH = [
    pltpu.VMEM((EPT,), jnp.int32),
    pltpu.VMEM((EPT,), jnp.int32),
    pltpu.VMEM((CH, HH), F32),
    pltpu.VMEM((CH, HH), F32),
    pltpu.VMEM((ZR, HH), F32),
    pltpu.VMEM((CH, HH), F32),
    pltpu.VMEM_SHARED((LP, HH), F32),
    pltpu.SemaphoreType.DMA,
    pltpu.SemaphoreType.DMA,
    pltpu.SemaphoreType.DMA,
    pltpu.SemaphoreType.DMA,
]
_SEG1_SCRATCH = _SEG_SCRATCH + [
    pltpu.VMEM_SHARED((LP, HH), F32),
]


def _phase_maker(idx_all, dst_all, rows, zrow_v, ones_v, acc, cnt, gsem,
                 ssem, cid, tid):
    def phase(src_h, dst_h, tab_h, out_h, cnt_h, rpt, counts):
        rbase = tid * rpt
        for z in range(rpt // ZR):
            pltpu.sync_copy(zrow_v, acc.at[pl.ds(rbase + z * ZR, ZR)])
            if counts:
                pltpu.sync_copy(zrow_v, cnt.at[pl.ds(rbase + z * ZR, ZR)])
        plsc.subcore_barrier()
        ebase = cid * EP + tid * EPT
        gd = [None] * NCH
        sd = [None] * NCH
        pltpu.sync_copy(src_h.at[pl.ds(ebase, EPT)], idx_all)
        pltpu.sync_copy(dst_h.at[pl.ds(ebase, EPT)], dst_all)
        ic = lambda k: idx_all.at[pl.ds(k * CH, CH)]
        dc = lambda k: dst_all.at[pl.ds(k * CH, CH)]
        gd[0] = pltpu.async_copy(tab_h.at[ic(0)], rows[0], gsem[0])
        for k in range(NCH):
            b = k & 1
            nb = 1 - b
            if k + 1 < NCH:
                if k >= 1:
                    sd[k - 1].wait()
                gd[k + 1] = pltpu.async_copy(tab_h.at[ic(k + 1)], rows[nb],
                                             gsem[nb])
            gd[k].wait()
            sd[k] = pltpu.async_copy(rows[b], acc.at[dc(k)], ssem[b],
                                     add=True)
            if counts:
                pltpu.sync_copy(ones_v, cnt.at[dc(k)], add=True)
        sd[NCH - 1].wait()
        if NCH >= 2:
            sd[NCH - 2].wait()
        plsc.subcore_barrier()
        obase = cid * rpt * NT + rbase
        pltpu.sync_copy(acc.at[pl.ds(rbase, rpt)],
                        out_h.at[pl.ds(obase, rpt)])
        if counts:
            pltpu.sync_copy(cnt.at[pl.ds(rbase, rpt)],
                            cnt_h.at[pl.ds(obase, rpt)])
        plsc.subcore_barrier()

    return phase


def _seg1_kernel():
    out_type = (
        jax.ShapeDtypeStruct((2 * LP, HH), F32),
        jax.ShapeDtypeStruct((2 * LP, HH), F32),
        jax.ShapeDtypeStruct((2 * AP, HH), F32),
        jax.ShapeDtypeStruct((2 * AP, HH), F32),
        jax.ShapeDtypeStruct((2 * LP, HH), F32),
        jax.ShapeDtypeStruct((2 * AP, HH), F32),
    )

    def body(src_b, dst_b, src_s, dst_s, tab_bl, tab_bh, tab_sl, tab_sh,
             zrow_h, ones_h,
             out_bl, out_bh, out_sl, out_sh, cnt_b, cnt_s,
             idx_all, dst_all, rows0, rows1, zrow_v, ones_v, acc,
             gsem0, gsem1, ssem0, ssem1, cnt):
        cid = lax.axis_index("c")
        tid = lax.axis_index("s")
        pltpu.sync_copy(zrow_h, zrow_v)
        pltpu.sync_copy(ones_h, ones_v)
        phase = _phase_maker(idx_all, dst_all, (rows0, rows1),
                             zrow_v, ones_v, acc, cnt,
                             (gsem0, gsem1), (ssem0, ssem1), cid, tid)
        phase(src_b, dst_b, tab_bl, out_bl, cnt_b, LP // NT, True)
        phase(src_b, dst_b, tab_bh, out_bh, None, LP // NT, False)
        phase(src_s, dst_s, tab_sl, out_sl, cnt_s, AP // NT, True)
        phase(src_s, dst_s, tab_sh, out_sh, None, AP // NT, False)

    return pl.kernel(body, out_type=out_type, mesh=_MESH,
                     scratch_types=_SEG1_SCRATCH)


def _seg2_kernel():
    out_type = (
        jax.ShapeDtypeStruct((2 * LP, HH), F32),
        jax.ShapeDtypeStruct((2 * AP, HH), F32),
    )

    def body(src_b, dst_b, src_s, dst_s, tab_b, tab_s, zrow_h,
             out_b, out_s,
             idx_all, dst_all, rows0, rows1, zrow_v, ones_v, acc,
             gsem0, gsem1, ssem0, ssem1):
        cid = lax.axis_index("c")
        tid = lax.axis_index("s")
        pltpu.sync_copy(zrow_h, zrow_v)
        phase = _phase_maker(idx_all, dst_all, (rows0, rows1),
                             zrow_v, ones_v, acc, None,
                             (gsem0, gsem1), (ssem0, ssem1), cid, tid)
        phase(src_b, dst_b, tab_b, out_b, None, LP // NT, False)
        phase(src_s, dst_s, tab_s, out_s, None, AP // NT, False)

    return pl.kernel(body, out_type=out_type, mesh=_MESH,
                     scratch_types=_SEG_SCRATCH)


def _tc1_livro(s2l, s2h, c2, s4l, s4h, c4, w2, w4, b2, b4, wl1, wl3, wr2, wr4):
    BLK = 256

    def body(s2lr, s2hr, c2r, s4lr, s4hr, c4r, w2r, w4r, b2r, b4r,
             a1r, a3r, r2r, r4r, p1o, p3o, rlo):
        cc2 = jnp.maximum(c2r[:, 0:1], 1.0)
        cc4 = jnp.maximum(c4r[:, 0:1], 1.0)
        s2 = jnp.concatenate([s2lr[...], s2hr[...]], axis=1)
        s4 = jnp.concatenate([s4lr[...], s4hr[...]], axis=1)
        h = (s2 / cc2 + b2r[...] + w2r[...] + s4 / cc4 + b4r[...] + w4r[...]) * 0.5
        h = jnp.maximum(h, 0.0)
        p1o[...] = jnp.dot(h, a1r[...], preferred_element_type=F32)
        p3o[...] = jnp.dot(h, a3r[...], preferred_element_type=F32)
        rlo[...] = jnp.dot(h, r2r[...] + r4r[...], preferred_element_type=F32)

    row = lambda i: (i, 0)
    fix = lambda i: (0, 0)
    return pl.pallas_call(
        body,
        grid=(LP // BLK,),
        in_specs=[
            pl.BlockSpec((BLK, HH), row), pl.BlockSpec((BLK, HH), row),
            pl.BlockSpec((BLK, HH), row),
            pl.BlockSpec((BLK, HH), row), pl.BlockSpec((BLK, HH), row),
            pl.BlockSpec((BLK, HH), row),
            pl.BlockSpec((BLK, HID), row), pl.BlockSpec((BLK, HID), row),
            pl.BlockSpec((1, HID), fix), pl.BlockSpec((1, HID), fix),
            pl.BlockSpec((HID, OUTD), fix), pl.BlockSpec((HID, OUTD), fix),
            pl.BlockSpec((HID, OUTD), fix), pl.BlockSpec((HID, OUTD), fix),
        ],
        out_specs=[pl.BlockSpec((BLK, OUTD), row)] * 3,
        out_shape=[jax.ShapeDtypeStruct((LP, OUTD), F32)] * 3,
    )(s2l, s2h, c2, s4l, s4h, c4, w2, w4, b2, b4, wl1, wl3, wr2, wr4)


def _tc1_single(s1l, s1h, c1, w1, b1, wl, wr, rows, blk):
    def body(s1lr, s1hr, c1r, w1r, b1r, wlr, wrr, po, ro):
        cc = jnp.maximum(c1r[:, 0:1], 1.0)
        s1 = jnp.concatenate([s1lr[...], s1hr[...]], axis=1)
        h = jnp.maximum(s1 / cc + b1r[...] + w1r[...], 0.0)
        po[...] = jnp.dot(h, wlr[...], preferred_element_type=F32)
        ro[...] = jnp.dot(h, wrr[...], preferred_element_type=F32)

    row = lambda i: (i, 0)
    fix = lambda i: (0, 0)
    return pl.pallas_call(
        body,
        grid=(rows // blk,),
        in_specs=[
            pl.BlockSpec((blk, HH), row), pl.BlockSpec((blk, HH), row),
            pl.BlockSpec((blk, HH), row),
            pl.BlockSpec((blk, HID), row), pl.BlockSpec((1, HID), fix),
            pl.BlockSpec((HID, OUTD), fix), pl.BlockSpec((HID, OUTD), fix),
        ],
        out_specs=[pl.BlockSpec((blk, OUTD), row)] * 2,
        out_shape=[jax.ShapeDtypeStruct((rows, OUTD), F32)] * 2,
    )(s1l, s1h, c1, w1, b1, wl, wr)


def _l2n(v):
    n = jnp.sqrt(jnp.sum(v * v, axis=1, keepdims=True))
    return v / jnp.maximum(n, 1e-12)


def _tc2_livro(s2, c2, s4, c4, rl, b2, b4):
    BLK = 256

    def body(s2r, c2r, s4r, c4r, rlr, b2r, b4r, o):
        cc2 = jnp.maximum(c2r[:, 0:1], 1.0)
        cc4 = jnp.maximum(c4r[:, 0:1], 1.0)
        v = (s2r[...] / cc2 + s4r[...] / cc4 + b2r[...] + b4r[...]
             + rlr[...]) * 0.5
        o[...] = _l2n(v)

    row = lambda i: (i, 0)
    fix = lambda i: (0, 0)
    return pl.pallas_call(
        body,
        grid=(LP // BLK,),
        in_specs=[
            pl.BlockSpec((BLK, OUTD), row), pl.BlockSpec((BLK, HH), row),
            pl.BlockSpec((BLK, OUTD), row), pl.BlockSpec((BLK, HH), row),
            pl.BlockSpec((BLK, OUTD), row),
            pl.BlockSpec((1, OUTD), fix), pl.BlockSpec((1, OUTD), fix),
        ],
        out_specs=pl.BlockSpec((BLK, OUTD), row),
        out_shape=jax.ShapeDtypeStruct((LP, OUTD), F32),
    )(s2, c2, s4, c4, rl, b2, b4)


def _tc2_single(s1, c1, r1, b1, rows, blk):
    def body(s1r, c1r, r1r, b1r, o):
        cc = jnp.maximum(c1r[:, 0:1], 1.0)
        o[...] = _l2n(s1r[...] / cc + b1r[...] + r1r[...])

    row = lambda i: (i, 0)
    fix = lambda i: (0, 0)
    return pl.pallas_call(
        body,
        grid=(rows // blk,),
        in_specs=[
            pl.BlockSpec((blk, OUTD), row), pl.BlockSpec((blk, HH), row),
            pl.BlockSpec((blk, OUTD), row), pl.BlockSpec((1, OUTD), fix),
        ],
        out_specs=pl.BlockSpec((blk, OUTD), row),
        out_shape=jax.ShapeDtypeStruct((rows, OUTD), F32),
    )(s1, c1, r1, b1)


def kernel(src_escrito_por, dst_escrito_por, src_escreveu, dst_escreveu,
           src_tem_genero, dst_tem_genero, src_pertence_a, dst_pertence_a,
           Wl1_e1, b1_e1, Wr1_e1, Wl1_e2, b1_e2, Wr1_e2,
           Wl1_e3, b1_e3, Wr1_e3, Wl1_e4, b1_e4, Wr1_e4,
           Wl2_e1, b2_e1, Wr2_e1, Wl2_e2, b2_e2, Wr2_e2,
           Wl2_e3, b2_e3, Wr2_e3, Wl2_e4, b2_e4, Wr2_e4):
    i32 = jnp.int32

    def pad_e(a, fill):
        return jnp.concatenate([a, jnp.full((EP - NE,), fill, i32)])

    s1p, d1p = pad_e(src_escrito_por, 0), pad_e(dst_escrito_por, NA)
    s2p, d2p = pad_e(src_escreveu, 0), pad_e(dst_escreveu, NL)
    s3p, d3p = pad_e(src_tem_genero, 0), pad_e(dst_tem_genero, NG)
    s4p, d4p = pad_e(src_pertence_a, 0), pad_e(dst_pertence_a, NL)

    src_b1 = jnp.concatenate([s2p, s4p + NA])
    dst_b = jnp.concatenate([d2p, d4p])
    src_s1 = jnp.concatenate([s1p, s3p + NL])
    dst_s = jnp.concatenate([d1p, d3p])
    src_b2 = jnp.concatenate([s2p, s4p + AP])
    src_s2 = jnp.concatenate([s1p, s3p + LP])

    zrow = jnp.zeros((ZR, HH), F32)
    ones = jnp.ones((CH, HH), F32)

    tab_bl = jnp.concatenate([Wl1_e2[:, :HH], Wl1_e4[:, :HH]])
    tab_bh = jnp.concatenate([Wl1_e2[:, HH:], Wl1_e4[:, HH:]])
    tab_sl = jnp.concatenate([Wl1_e1[:, :HH], Wl1_e3[:, :HH]])
    tab_sh = jnp.concatenate([Wl1_e1[:, HH:], Wl1_e3[:, HH:]])
    out_bl, out_bh, out_sl, out_sh, cnt_b, cnt_s = _seg1_kernel()(
        src_b1, dst_b, src_s1, dst_s, tab_bl, tab_bh, tab_sl, tab_sh,
        zrow, ones)
    cnt_e2, cnt_e4 = cnt_b[:LP], cnt_b[LP:]
    cnt_e1, cnt_e3 = cnt_s[:AP], cnt_s[AP:]

    wr1_e2p = jnp.pad(Wr1_e2, ((0, LP - NL), (0, 0)))
    wr1_e4p = jnp.pad(Wr1_e4, ((0, LP - NL), (0, 0)))
    wr1_e1p = jnp.pad(Wr1_e1, ((0, AP - NA), (0, 0)))
    wr1_e3p = jnp.pad(Wr1_e3, ((0, GP - NG), (0, 0)))
    b = lambda x: x.reshape(1, -1)

    P1, P3, RL = _tc1_livro(out_bl[:LP], out_bh[:LP], cnt_e2,
                            out_bl[LP:], out_bh[LP:], cnt_e4,
                            wr1_e2p, wr1_e4p, b(b1_e2), b(b1_e4),
                            Wl2_e1, Wl2_e3, Wr2_e2, Wr2_e4)
    P2, R1 = _tc1_single(out_sl[:AP], out_sh[:AP], cnt_e1, wr1_e1p, b(b1_e1),
                         Wl2_e2, Wr2_e1, AP, 256)
    P4, R3 = _tc1_single(out_sl[AP:AP + GP], out_sh[AP:AP + GP], cnt_e3[:GP],
                         wr1_e3p, b(b1_e3), Wl2_e4, Wr2_e3, GP, GP)

    tab_b2 = jnp.concatenate([P2, P4])
    tab_s2 = jnp.concatenate([P1, P3])
    out2_b, out2_s = _seg2_kernel()(
        src_b2, dst_b, src_s2, dst_s, tab_b2, tab_s2, zrow)

    out_l = _tc2_livro(out2_b[:LP], cnt_e2, out2_b[LP:], cnt_e4, RL,
                       b(b2_e2), b(b2_e4))
    out_a = _tc2_single(out2_s[:AP], cnt_e1, R1, b(b2_e1), AP, 256)
    out_g = _tc2_single(out2_s[AP:AP + GP], cnt_e3[:GP], R3, b(b2_e3), GP, GP)

    return (out_l[:NL], out_a[:NA], out_g[:NG])


# speedup vs baseline: 7.0252x; 1.0308x over previous
import jax
import jax.numpy as jnp
from jax import lax
from jax.experimental import pallas as pl
from jax.experimental.pallas import tpu as pltpu
from jax.experimental.pallas import tpu_sc as plsc

NL, NA, NG = 5000, 2500, 100
NE = 10000
HID, OUTD = 256, 128
HH = 128
LP, AP, GP = 5120, 2560, 128
EP = 10240
NT = 16
EPT = EP // NT
CH = 64
NCH = EPT // CH
ZR = 80
CW = 16
F32 = jnp.float32

_MESH = plsc.VectorSubcoreMesh(core_axis_name="c", subcore_axis_name="s")

_SEG_SCRATCH = [
    pltpu.VMEM((EPT,), jnp.int32),
    pltpu.VMEM((EPT,), jnp.int32),
    pltpu.VMEM((CH, HH), F32),
    pltpu.VMEM((CH, HH), F32),
    pltpu.VMEM((ZR, HH), F32),
    pltpu.VMEM((CH, HH), F32),
    pltpu.VMEM_SHARED((LP, HH), F32),
    pltpu.SemaphoreType.DMA,
    pltpu.SemaphoreType.DMA,
    pltpu.SemaphoreType.DMA,
    pltpu.SemaphoreType.DMA,
]
_SEG1_SCRATCH = _SEG_SCRATCH + [
    pltpu.VMEM_SHARED((LP, HH), F32),
]


def _phase_maker(idx_all, dst_all, rows, zrow_v, ones_v, acc, cnt, gsem,
                 ssem, cid, tid):
    def phase(src_h, dst_h, tab_h, out_h, cnt_h, rpt, counts):
        rbase = tid * rpt
        for z in range(rpt // ZR):
            pltpu.sync_copy(zrow_v, acc.at[pl.ds(rbase + z * ZR, ZR)])
            if counts:
                pltpu.sync_copy(zrow_v, cnt.at[pl.ds(rbase + z * ZR, ZR)])
        plsc.subcore_barrier()
        ebase = cid * EP + tid * EPT
        gd = [None] * NCH
        sd = [None] * NCH
        pltpu.sync_copy(src_h.at[pl.ds(ebase, EPT)], idx_all)
        pltpu.sync_copy(dst_h.at[pl.ds(ebase, EPT)], dst_all)
        ic = lambda k: idx_all.at[pl.ds(k * CH, CH)]
        dc = lambda k: dst_all.at[pl.ds(k * CH, CH)]
        gd[0] = pltpu.async_copy(tab_h.at[ic(0)], rows[0], gsem[0])
        for k in range(NCH):
            b = k & 1
            nb = 1 - b
            if k + 1 < NCH:
                if k >= 1:
                    sd[k - 1].wait()
                gd[k + 1] = pltpu.async_copy(tab_h.at[ic(k + 1)], rows[nb],
                                             gsem[nb])
            gd[k].wait()
            sd[k] = pltpu.async_copy(rows[b], acc.at[dc(k)], ssem[b],
                                     add=True)
            if counts:
                pltpu.sync_copy(ones_v, cnt.at[dc(k)], add=True)
        sd[NCH - 1].wait()
        if NCH >= 2:
            sd[NCH - 2].wait()
        plsc.subcore_barrier()
        obase = cid * rpt * NT + rbase
        pltpu.sync_copy(acc.at[pl.ds(rbase, rpt)],
                        out_h.at[pl.ds(obase, rpt)])
        if counts:
            pltpu.sync_copy(cnt.at[pl.ds(rbase, rpt)],
                            cnt_h.at[pl.ds(obase, rpt)])
        plsc.subcore_barrier()

    return phase


def _seg1_kernel():
    out_type = (
        jax.ShapeDtypeStruct((2 * LP, HH), F32),
        jax.ShapeDtypeStruct((2 * LP, HH), F32),
        jax.ShapeDtypeStruct((2 * AP, HH), F32),
        jax.ShapeDtypeStruct((2 * AP, HH), F32),
        jax.ShapeDtypeStruct((2 * LP, HH), F32),
        jax.ShapeDtypeStruct((2 * AP, HH), F32),
    )

    def body(src_b, dst_b, src_s, dst_s, tab_bl, tab_bh, tab_sl, tab_sh,
             zrow_h, ones_h,
             out_bl, out_bh, out_sl, out_sh, cnt_b, cnt_s,
             idx_all, dst_all, rows0, rows1, zrow_v, ones_v, acc,
             gsem0, gsem1, ssem0, ssem1, cnt):
        cid = lax.axis_index("c")
        tid = lax.axis_index("s")
        pltpu.sync_copy(zrow_h, zrow_v)
        pltpu.sync_copy(ones_h, ones_v)
        phase = _phase_maker(idx_all, dst_all, (rows0, rows1),
                             zrow_v, ones_v, acc, cnt,
                             (gsem0, gsem1), (ssem0, ssem1), cid, tid)
        phase(src_b, dst_b, tab_bl, out_bl, cnt_b, LP // NT, True)
        phase(src_b, dst_b, tab_bh, out_bh, None, LP // NT, False)
        phase(src_s, dst_s, tab_sl, out_sl, cnt_s, AP // NT, True)
        phase(src_s, dst_s, tab_sh, out_sh, None, AP // NT, False)

    return pl.kernel(body, out_type=out_type, mesh=_MESH,
                     scratch_types=_SEG1_SCRATCH)


def _seg2_kernel():
    out_type = (
        jax.ShapeDtypeStruct((2 * LP, HH), F32),
        jax.ShapeDtypeStruct((2 * AP, HH), F32),
    )

    def body(src_b, dst_b, src_s, dst_s, tab_b, tab_s, zrow_h,
             out_b, out_s,
             idx_all, dst_all, rows0, rows1, zrow_v, ones_v, acc,
             gsem0, gsem1, ssem0, ssem1):
        cid = lax.axis_index("c")
        tid = lax.axis_index("s")
        pltpu.sync_copy(zrow_h, zrow_v)
        phase = _phase_maker(idx_all, dst_all, (rows0, rows1),
                             zrow_v, ones_v, acc, None,
                             (gsem0, gsem1), (ssem0, ssem1), cid, tid)
        phase(src_b, dst_b, tab_b, out_b, None, LP // NT, False)
        phase(src_s, dst_s, tab_s, out_s, None, AP // NT, False)

    return pl.kernel(body, out_type=out_type, mesh=_MESH,
                     scratch_types=_SEG_SCRATCH)


def _tc1_livro(s2l, s2h, c2, s4l, s4h, c4, w2, w4, b2, b4, wl1, wl3, wr2, wr4):
    BLK = 256

    def body(s2lr, s2hr, c2r, s4lr, s4hr, c4r, w2r, w4r, b2r, b4r,
             a1r, a3r, r2r, r4r, p1o, p3o, rlo):
        cc2 = jnp.maximum(c2r[:, 0:1], 1.0)
        cc4 = jnp.maximum(c4r[:, 0:1], 1.0)
        s2 = jnp.concatenate([s2lr[...], s2hr[...]], axis=1)
        s4 = jnp.concatenate([s4lr[...], s4hr[...]], axis=1)
        h = (s2 / cc2 + b2r[...] + w2r[...] + s4 / cc4 + b4r[...] + w4r[...]) * 0.5
        h = jnp.maximum(h, 0.0)
        p1o[...] = jnp.dot(h, a1r[...], preferred_element_type=F32)
        p3o[...] = jnp.dot(h, a3r[...], preferred_element_type=F32)
        rlo[...] = jnp.dot(h, r2r[...] + r4r[...], preferred_element_type=F32)

    row = lambda i: (i, 0)
    fix = lambda i: (0, 0)
    return pl.pallas_call(
        body,
        grid=(LP // BLK,),
        in_specs=[
            pl.BlockSpec((BLK, HH), row), pl.BlockSpec((BLK, HH), row),
            pl.BlockSpec((BLK, HH), row),
            pl.BlockSpec((BLK, HH), row), pl.BlockSpec((BLK, HH), row),
            pl.BlockSpec((BLK, HH), row),
            pl.BlockSpec((BLK, HID), row), pl.BlockSpec((BLK, HID), row),
            pl.BlockSpec((1, HID), fix), pl.BlockSpec((1, HID), fix),
            pl.BlockSpec((HID, OUTD), fix), pl.BlockSpec((HID, OUTD), fix),
            pl.BlockSpec((HID, OUTD), fix), pl.BlockSpec((HID, OUTD), fix),
        ],
        out_specs=[pl.BlockSpec((BLK, OUTD), row)] * 3,
        out_shape=[jax.ShapeDtypeStruct((LP, OUTD), F32)] * 3,
    )(s2l, s2h, c2, s4l, s4h, c4, w2, w4, b2, b4, wl1, wl3, wr2, wr4)


def _tc1_single(s1l, s1h, c1, w1, b1, wl, wr, rows, blk):
    def body(s1lr, s1hr, c1r, w1r, b1r, wlr, wrr, po, ro):
        cc = jnp.maximum(c1r[:, 0:1], 1.0)
        s1 = jnp.concatenate([s1lr[...], s1hr[...]], axis=1)
        h = jnp.maximum(s1 / cc + b1r[...] + w1r[...], 0.0)
        po[...] = jnp.dot(h, wlr[...], preferred_element_type=F32)
        ro[...] = jnp.dot(h, wrr[...], preferred_element_type=F32)

    row = lambda i: (i, 0)
    fix = lambda i: (0, 0)
    return pl.pallas_call(
        body,
        grid=(rows // blk,),
        in_specs=[
            pl.BlockSpec((blk, HH), row), pl.BlockSpec((blk, HH), row),
            pl.BlockSpec((blk, HH), row),
            pl.BlockSpec((blk, HID), row), pl.BlockSpec((1, HID), fix),
            pl.BlockSpec((HID, OUTD), fix), pl.BlockSpec((HID, OUTD), fix),
        ],
        out_specs=[pl.BlockSpec((blk, OUTD), row)] * 2,
        out_shape=[jax.ShapeDtypeStruct((rows, OUTD), F32)] * 2,
    )(s1l, s1h, c1, w1, b1, wl, wr)


def _l2n(v):
    n = jnp.sqrt(jnp.sum(v * v, axis=1, keepdims=True))
    return v / jnp.maximum(n, 1e-12)


def _tc2_livro(s2, c2, s4, c4, rl, b2, b4, rows):
    BLK = 256

    def body(s2r, c2r, s4r, c4r, rlr, b2r, b4r, o):
        cc2 = jnp.maximum(c2r[:, 0:1], 1.0)
        cc4 = jnp.maximum(c4r[:, 0:1], 1.0)
        v = (s2r[...] / cc2 + s4r[...] / cc4 + b2r[...] + b4r[...]
             + rlr[...]) * 0.5
        o[...] = _l2n(v)

    row = lambda i: (i, 0)
    fix = lambda i: (0, 0)
    return pl.pallas_call(
        body,
        grid=(LP // BLK,),
        in_specs=[
            pl.BlockSpec((BLK, OUTD), row), pl.BlockSpec((BLK, HH), row),
            pl.BlockSpec((BLK, OUTD), row), pl.BlockSpec((BLK, HH), row),
            pl.BlockSpec((BLK, OUTD), row),
            pl.BlockSpec((1, OUTD), fix), pl.BlockSpec((1, OUTD), fix),
        ],
        out_specs=pl.BlockSpec((BLK, OUTD), row),
        out_shape=jax.ShapeDtypeStruct((rows, OUTD), F32),
    )(s2, c2, s4, c4, rl, b2, b4)


def _tc2_single(s1, c1, r1, b1, rows, blk, orows):
    def body(s1r, c1r, r1r, b1r, o):
        cc = jnp.maximum(c1r[:, 0:1], 1.0)
        o[...] = _l2n(s1r[...] / cc + b1r[...] + r1r[...])

    row = lambda i: (i, 0)
    fix = lambda i: (0, 0)
    return pl.pallas_call(
        body,
        grid=(rows // blk,),
        in_specs=[
            pl.BlockSpec((blk, OUTD), row), pl.BlockSpec((blk, HH), row),
            pl.BlockSpec((blk, OUTD), row), pl.BlockSpec((1, OUTD), fix),
        ],
        out_specs=pl.BlockSpec((blk, OUTD), row),
        out_shape=jax.ShapeDtypeStruct((orows, OUTD), F32),
    )(s1, c1, r1, b1)


def kernel(src_escrito_por, dst_escrito_por, src_escreveu, dst_escreveu,
           src_tem_genero, dst_tem_genero, src_pertence_a, dst_pertence_a,
           Wl1_e1, b1_e1, Wr1_e1, Wl1_e2, b1_e2, Wr1_e2,
           Wl1_e3, b1_e3, Wr1_e3, Wl1_e4, b1_e4, Wr1_e4,
           Wl2_e1, b2_e1, Wr2_e1, Wl2_e2, b2_e2, Wr2_e2,
           Wl2_e3, b2_e3, Wr2_e3, Wl2_e4, b2_e4, Wr2_e4):
    i32 = jnp.int32

    def pad_e(a, fill):
        return jnp.concatenate([a, jnp.full((EP - NE,), fill, i32)])

    s1p, d1p = pad_e(src_escrito_por, 0), pad_e(dst_escrito_por, NA)
    s2p, d2p = pad_e(src_escreveu, 0), pad_e(dst_escreveu, NL)
    s3p, d3p = pad_e(src_tem_genero, 0), pad_e(dst_tem_genero, NG)
    s4p, d4p = pad_e(src_pertence_a, 0), pad_e(dst_pertence_a, NL)

    src_b1 = jnp.concatenate([s2p, s4p + NA])
    dst_b = jnp.concatenate([d2p, d4p])
    src_s1 = jnp.concatenate([s1p, s3p + NL])
    dst_s = jnp.concatenate([d1p, d3p])
    src_b2 = jnp.concatenate([s2p, s4p + AP])
    src_s2 = jnp.concatenate([s1p, s3p + LP])

    zrow = jnp.zeros((ZR, HH), F32)
    ones = jnp.ones((CH, HH), F32)

    tab_bl = jnp.concatenate([Wl1_e2[:, :HH], Wl1_e4[:, :HH]])
    tab_bh = jnp.concatenate([Wl1_e2[:, HH:], Wl1_e4[:, HH:]])
    tab_sl = jnp.concatenate([Wl1_e1[:, :HH], Wl1_e3[:, :HH]])
    tab_sh = jnp.concatenate([Wl1_e1[:, HH:], Wl1_e3[:, HH:]])
    out_bl, out_bh, out_sl, out_sh, cnt_b, cnt_s = _seg1_kernel()(
        src_b1, dst_b, src_s1, dst_s, tab_bl, tab_bh, tab_sl, tab_sh,
        zrow, ones)
    cnt_e2, cnt_e4 = cnt_b[:LP], cnt_b[LP:]
    cnt_e1, cnt_e3 = cnt_s[:AP], cnt_s[AP:]

    b = lambda x: x.reshape(1, -1)

    P1, P3, RL = _tc1_livro(out_bl[:LP], out_bh[:LP], cnt_e2,
                            out_bl[LP:], out_bh[LP:], cnt_e4,
                            Wr1_e2, Wr1_e4, b(b1_e2), b(b1_e4),
                            Wl2_e1, Wl2_e3, Wr2_e2, Wr2_e4)
    P2, R1 = _tc1_single(out_sl[:AP], out_sh[:AP], cnt_e1, Wr1_e1, b(b1_e1),
                         Wl2_e2, Wr2_e1, AP, 256)
    P4, R3 = _tc1_single(out_sl[AP:AP + GP], out_sh[AP:AP + GP], cnt_e3[:GP],
                         Wr1_e3, b(b1_e3), Wl2_e4, Wr2_e3, GP, GP)

    tab_b2 = jnp.concatenate([P2, P4])
    tab_s2 = jnp.concatenate([P1, P3])
    out2_b, out2_s = _seg2_kernel()(
        src_b2, dst_b, src_s2, dst_s, tab_b2, tab_s2, zrow)

    out_l = _tc2_livro(out2_b[:LP], cnt_e2, out2_b[LP:], cnt_e4, RL,
                       b(b2_e2), b(b2_e4), NL)
    out_a = _tc2_single(out2_s[:AP], cnt_e1, R1, b(b2_e1), AP, 256, NA)
    out_g = _tc2_single(out2_s[AP:AP + GP], cnt_e3[:GP], R3, b(b2_e3),
                        GP, GP, NG)

    return (out_l, out_a, out_g)


# speedup vs baseline: 7.0781x; 1.0075x over previous
import jax
import jax.numpy as jnp
from jax import lax
from jax.experimental import pallas as pl
from jax.experimental.pallas import tpu as pltpu
from jax.experimental.pallas import tpu_sc as plsc

NL, NA, NG = 5000, 2500, 100
NE = 10000
HID, OUTD = 256, 128
HH = 128
LP, AP, GP = 5120, 2560, 128
EP = 10240
NT = 16
EPT = EP // NT
CH = 80
NCH = EPT // CH
ZR = 80
CW = 16
F32 = jnp.float32

_MESH = plsc.VectorSubcoreMesh(core_axis_name="c", subcore_axis_name="s")

_SEG_SCRATCH = [
    pltpu.VMEM((EPT,), jnp.int32),
    pltpu.VMEM((EPT,), jnp.int32),
    pltpu.VMEM((CH, HH), F32),
    pltpu.VMEM((CH, HH), F32),
    pltpu.VMEM((ZR, HH), F32),
    pltpu.VMEM((CH, HH), F32),
    pltpu.VMEM_SHARED((LP, HH), F32),
    pltpu.SemaphoreType.DMA,
    pltpu.SemaphoreType.DMA,
    pltpu.SemaphoreType.DMA,
    pltpu.SemaphoreType.DMA,
]
_SEG1_SCRATCH = _SEG_SCRATCH + [
    pltpu.VMEM_SHARED((LP, HH), F32),
]


def _phase_maker(idx_all, dst_all, rows, zrow_v, ones_v, acc, cnt, gsem,
                 ssem, cid, tid):
    def phase(src_h, dst_h, tab_h, out_h, cnt_h, rpt, counts):
        rbase = tid * rpt
        for z in range(rpt // ZR):
            pltpu.sync_copy(zrow_v, acc.at[pl.ds(rbase + z * ZR, ZR)])
            if counts:
                pltpu.sync_copy(zrow_v, cnt.at[pl.ds(rbase + z * ZR, ZR)])
        plsc.subcore_barrier()
        ebase = cid * EP + tid * EPT
        gd = [None] * NCH
        sd = [None] * NCH
        pltpu.sync_copy(src_h.at[pl.ds(ebase, EPT)], idx_all)
        pltpu.sync_copy(dst_h.at[pl.ds(ebase, EPT)], dst_all)
        ic = lambda k: idx_all.at[pl.ds(k * CH, CH)]
        dc = lambda k: dst_all.at[pl.ds(k * CH, CH)]
        gd[0] = pltpu.async_copy(tab_h.at[ic(0)], rows[0], gsem[0])
        for k in range(NCH):
            b = k & 1
            nb = 1 - b
            if k + 1 < NCH:
                if k >= 1:
                    sd[k - 1].wait()
                gd[k + 1] = pltpu.async_copy(tab_h.at[ic(k + 1)], rows[nb],
                                             gsem[nb])
            gd[k].wait()
            sd[k] = pltpu.async_copy(rows[b], acc.at[dc(k)], ssem[b],
                                     add=True)
            if counts:
                pltpu.sync_copy(ones_v, cnt.at[dc(k)], add=True)
        sd[NCH - 1].wait()
        if NCH >= 2:
            sd[NCH - 2].wait()
        plsc.subcore_barrier()
        obase = cid * rpt * NT + rbase
        pltpu.sync_copy(acc.at[pl.ds(rbase, rpt)],
                        out_h.at[pl.ds(obase, rpt)])
        if counts:
            pltpu.sync_copy(cnt.at[pl.ds(rbase, rpt)],
                            cnt_h.at[pl.ds(obase, rpt)])
        plsc.subcore_barrier()

    return phase


def _seg1_kernel():
    out_type = (
        jax.ShapeDtypeStruct((2 * LP, HH), F32),
        jax.ShapeDtypeStruct((2 * LP, HH), F32),
        jax.ShapeDtypeStruct((2 * AP, HH), F32),
        jax.ShapeDtypeStruct((2 * AP, HH), F32),
        jax.ShapeDtypeStruct((2 * LP, HH), F32),
        jax.ShapeDtypeStruct((2 * AP, HH), F32),
    )

    def body(src_b, dst_b, src_s, dst_s, tab_bl, tab_bh, tab_sl, tab_sh,
             zrow_h, ones_h,
             out_bl, out_bh, out_sl, out_sh, cnt_b, cnt_s,
             idx_all, dst_all, rows0, rows1, zrow_v, ones_v, acc,
             gsem0, gsem1, ssem0, ssem1, cnt):
        cid = lax.axis_index("c")
        tid = lax.axis_index("s")
        pltpu.sync_copy(zrow_h, zrow_v)
        pltpu.sync_copy(ones_h, ones_v)
        phase = _phase_maker(idx_all, dst_all, (rows0, rows1),
                             zrow_v, ones_v, acc, cnt,
                             (gsem0, gsem1), (ssem0, ssem1), cid, tid)
        phase(src_b, dst_b, tab_bl, out_bl, cnt_b, LP // NT, True)
        phase(src_b, dst_b, tab_bh, out_bh, None, LP // NT, False)
        phase(src_s, dst_s, tab_sl, out_sl, cnt_s, AP // NT, True)
        phase(src_s, dst_s, tab_sh, out_sh, None, AP // NT, False)

    return pl.kernel(body, out_type=out_type, mesh=_MESH,
                     scratch_types=_SEG1_SCRATCH)


def _seg2_kernel():
    out_type = (
        jax.ShapeDtypeStruct((2 * LP, HH), F32),
        jax.ShapeDtypeStruct((2 * AP, HH), F32),
    )

    def body(src_b, dst_b, src_s, dst_s, tab_b, tab_s, zrow_h,
             out_b, out_s,
             idx_all, dst_all, rows0, rows1, zrow_v, ones_v, acc,
             gsem0, gsem1, ssem0, ssem1):
        cid = lax.axis_index("c")
        tid = lax.axis_index("s")
        pltpu.sync_copy(zrow_h, zrow_v)
        phase = _phase_maker(idx_all, dst_all, (rows0, rows1),
                             zrow_v, ones_v, acc, None,
                             (gsem0, gsem1), (ssem0, ssem1), cid, tid)
        phase(src_b, dst_b, tab_b, out_b, None, LP // NT, False)
        phase(src_s, dst_s, tab_s, out_s, None, AP // NT, False)

    return pl.kernel(body, out_type=out_type, mesh=_MESH,
                     scratch_types=_SEG_SCRATCH)


def _tc1_livro(s2l, s2h, c2, s4l, s4h, c4, w2, w4, b2, b4, wl1, wl3, wr2, wr4):
    BLK = 256

    def body(s2lr, s2hr, c2r, s4lr, s4hr, c4r, w2r, w4r, b2r, b4r,
             a1r, a3r, r2r, r4r, p1o, p3o, rlo):
        cc2 = jnp.maximum(c2r[:, 0:1], 1.0)
        cc4 = jnp.maximum(c4r[:, 0:1], 1.0)
        s2 = jnp.concatenate([s2lr[...], s2hr[...]], axis=1)
        s4 = jnp.concatenate([s4lr[...], s4hr[...]], axis=1)
        h = (s2 / cc2 + b2r[...] + w2r[...] + s4 / cc4 + b4r[...] + w4r[...]) * 0.5
        h = jnp.maximum(h, 0.0)
        p1o[...] = jnp.dot(h, a1r[...], preferred_element_type=F32)
        p3o[...] = jnp.dot(h, a3r[...], preferred_element_type=F32)
        rlo[...] = jnp.dot(h, r2r[...] + r4r[...], preferred_element_type=F32)

    row = lambda i: (i, 0)
    fix = lambda i: (0, 0)
    return pl.pallas_call(
        body,
        grid=(LP // BLK,),
        in_specs=[
            pl.BlockSpec((BLK, HH), row), pl.BlockSpec((BLK, HH), row),
            pl.BlockSpec((BLK, HH), row),
            pl.BlockSpec((BLK, HH), row), pl.BlockSpec((BLK, HH), row),
            pl.BlockSpec((BLK, HH), row),
            pl.BlockSpec((BLK, HID), row), pl.BlockSpec((BLK, HID), row),
            pl.BlockSpec((1, HID), fix), pl.BlockSpec((1, HID), fix),
            pl.BlockSpec((HID, OUTD), fix), pl.BlockSpec((HID, OUTD), fix),
            pl.BlockSpec((HID, OUTD), fix), pl.BlockSpec((HID, OUTD), fix),
        ],
        out_specs=[pl.BlockSpec((BLK, OUTD), row)] * 3,
        out_shape=[jax.ShapeDtypeStruct((LP, OUTD), F32)] * 3,
    )(s2l, s2h, c2, s4l, s4h, c4, w2, w4, b2, b4, wl1, wl3, wr2, wr4)


def _tc1_single(s1l, s1h, c1, w1, b1, wl, wr, rows, blk):
    def body(s1lr, s1hr, c1r, w1r, b1r, wlr, wrr, po, ro):
        cc = jnp.maximum(c1r[:, 0:1], 1.0)
        s1 = jnp.concatenate([s1lr[...], s1hr[...]], axis=1)
        h = jnp.maximum(s1 / cc + b1r[...] + w1r[...], 0.0)
        po[...] = jnp.dot(h, wlr[...], preferred_element_type=F32)
        ro[...] = jnp.dot(h, wrr[...], preferred_element_type=F32)

    row = lambda i: (i, 0)
    fix = lambda i: (0, 0)
    return pl.pallas_call(
        body,
        grid=(rows // blk,),
        in_specs=[
            pl.BlockSpec((blk, HH), row), pl.BlockSpec((blk, HH), row),
            pl.BlockSpec((blk, HH), row),
            pl.BlockSpec((blk, HID), row), pl.BlockSpec((1, HID), fix),
            pl.BlockSpec((HID, OUTD), fix), pl.BlockSpec((HID, OUTD), fix),
        ],
        out_specs=[pl.BlockSpec((blk, OUTD), row)] * 2,
        out_shape=[jax.ShapeDtypeStruct((rows, OUTD), F32)] * 2,
    )(s1l, s1h, c1, w1, b1, wl, wr)


def _l2n(v):
    n = jnp.sqrt(jnp.sum(v * v, axis=1, keepdims=True))
    return v / jnp.maximum(n, 1e-12)


def _tc2_livro(s2, c2, s4, c4, rl, b2, b4, rows):
    BLK = 256

    def body(s2r, c2r, s4r, c4r, rlr, b2r, b4r, o):
        cc2 = jnp.maximum(c2r[:, 0:1], 1.0)
        cc4 = jnp.maximum(c4r[:, 0:1], 1.0)
        v = (s2r[...] / cc2 + s4r[...] / cc4 + b2r[...] + b4r[...]
             + rlr[...]) * 0.5
        o[...] = _l2n(v)

    row = lambda i: (i, 0)
    fix = lambda i: (0, 0)
    return pl.pallas_call(
        body,
        grid=(LP // BLK,),
        in_specs=[
            pl.BlockSpec((BLK, OUTD), row), pl.BlockSpec((BLK, HH), row),
            pl.BlockSpec((BLK, OUTD), row), pl.BlockSpec((BLK, HH), row),
            pl.BlockSpec((BLK, OUTD), row),
            pl.BlockSpec((1, OUTD), fix), pl.BlockSpec((1, OUTD), fix),
        ],
        out_specs=pl.BlockSpec((BLK, OUTD), row),
        out_shape=jax.ShapeDtypeStruct((rows, OUTD), F32),
    )(s2, c2, s4, c4, rl, b2, b4)


def _tc2_single(s1, c1, r1, b1, rows, blk, orows):
    def body(s1r, c1r, r1r, b1r, o):
        cc = jnp.maximum(c1r[:, 0:1], 1.0)
        o[...] = _l2n(s1r[...] / cc + b1r[...] + r1r[...])

    row = lambda i: (i, 0)
    fix = lambda i: (0, 0)
    return pl.pallas_call(
        body,
        grid=(rows // blk,),
        in_specs=[
            pl.BlockSpec((blk, OUTD), row), pl.BlockSpec((blk, HH), row),
            pl.BlockSpec((blk, OUTD), row), pl.BlockSpec((1, OUTD), fix),
        ],
        out_specs=pl.BlockSpec((blk, OUTD), row),
        out_shape=jax.ShapeDtypeStruct((orows, OUTD), F32),
    )(s1, c1, r1, b1)


def kernel(src_escrito_por, dst_escrito_por, src_escreveu, dst_escreveu,
           src_tem_genero, dst_tem_genero, src_pertence_a, dst_pertence_a,
           Wl1_e1, b1_e1, Wr1_e1, Wl1_e2, b1_e2, Wr1_e2,
           Wl1_e3, b1_e3, Wr1_e3, Wl1_e4, b1_e4, Wr1_e4,
           Wl2_e1, b2_e1, Wr2_e1, Wl2_e2, b2_e2, Wr2_e2,
           Wl2_e3, b2_e3, Wr2_e3, Wl2_e4, b2_e4, Wr2_e4):
    i32 = jnp.int32

    def pad_e(a, fill):
        return jnp.concatenate([a, jnp.full((EP - NE,), fill, i32)])

    s1p, d1p = pad_e(src_escrito_por, 0), pad_e(dst_escrito_por, NA)
    s2p, d2p = pad_e(src_escreveu, 0), pad_e(dst_escreveu, NL)
    s3p, d3p = pad_e(src_tem_genero, 0), pad_e(dst_tem_genero, NG)
    s4p, d4p = pad_e(src_pertence_a, 0), pad_e(dst_pertence_a, NL)

    src_b1 = jnp.concatenate([s2p, s4p + NA])
    dst_b = jnp.concatenate([d2p, d4p])
    src_s1 = jnp.concatenate([s1p, s3p + NL])
    dst_s = jnp.concatenate([d1p, d3p])
    src_b2 = jnp.concatenate([s2p, s4p + AP])
    src_s2 = jnp.concatenate([s1p, s3p + LP])

    zrow = jnp.zeros((ZR, HH), F32)
    ones = jnp.ones((CH, HH), F32)

    tab_bl = jnp.concatenate([Wl1_e2[:, :HH], Wl1_e4[:, :HH]])
    tab_bh = jnp.concatenate([Wl1_e2[:, HH:], Wl1_e4[:, HH:]])
    tab_sl = jnp.concatenate([Wl1_e1[:, :HH], Wl1_e3[:, :HH]])
    tab_sh = jnp.concatenate([Wl1_e1[:, HH:], Wl1_e3[:, HH:]])
    out_bl, out_bh, out_sl, out_sh, cnt_b, cnt_s = _seg1_kernel()(
        src_b1, dst_b, src_s1, dst_s, tab_bl, tab_bh, tab_sl, tab_sh,
        zrow, ones)
    cnt_e2, cnt_e4 = cnt_b[:LP], cnt_b[LP:]
    cnt_e1, cnt_e3 = cnt_s[:AP], cnt_s[AP:]

    b = lambda x: x.reshape(1, -1)

    P1, P3, RL = _tc1_livro(out_bl[:LP], out_bh[:LP], cnt_e2,
                            out_bl[LP:], out_bh[LP:], cnt_e4,
                            Wr1_e2, Wr1_e4, b(b1_e2), b(b1_e4),
                            Wl2_e1, Wl2_e3, Wr2_e2, Wr2_e4)
    P2, R1 = _tc1_single(out_sl[:AP], out_sh[:AP], cnt_e1, Wr1_e1, b(b1_e1),
                         Wl2_e2, Wr2_e1, AP, 256)
    P4, R3 = _tc1_single(out_sl[AP:AP + GP], out_sh[AP:AP + GP], cnt_e3[:GP],
                         Wr1_e3, b(b1_e3), Wl2_e4, Wr2_e3, GP, GP)

    tab_b2 = jnp.concatenate([P2, P4])
    tab_s2 = jnp.concatenate([P1, P3])
    out2_b, out2_s = _seg2_kernel()(
        src_b2, dst_b, src_s2, dst_s, tab_b2, tab_s2, zrow)

    out_l = _tc2_livro(out2_b[:LP], cnt_e2, out2_b[LP:], cnt_e4, RL,
                       b(b2_e2), b(b2_e4), NL)
    out_a = _tc2_single(out2_s[:AP], cnt_e1, R1, b(b2_e1), AP, 256, NA)
    out_g = _tc2_single(out2_s[AP:AP + GP], cnt_e3[:GP], R3, b(b2_e3),
                        GP, GP, NG)

    return (out_l, out_a, out_g)
